# baseline (speedup 1.0000x reference)
import functools

import jax
import jax.numpy as jnp
from jax import lax
from jax.experimental import pallas as pl
from jax.experimental.pallas import tpu as pltpu
from jax.experimental.pallas import tpu_sc as plsc

B = 4096
S = 200
D = 64
L = 16

NC = 2
NS = 16
NW = NC * NS

NB = B // NW
CH_L = 2
NCHUNK = S // CH_L
NPAIR = NCHUNK // 2


def _sc_embed(xT, table2, pe2):
    mesh = plsc.VectorSubcoreMesh(
        core_axis_name="c", subcore_axis_name="s", num_cores=NC,
        num_subcores=NS)

    @functools.partial(
        pl.kernel,
        mesh=mesh,
        out_type=jax.ShapeDtypeStruct((S, 8, NW, 8, NB), jnp.float32),
        scratch_types=[
            [pltpu.VMEM((CH_L, NB), jnp.int32) for _ in range(2)],
            [pltpu.VMEM((CH_L, NB), jnp.int32) for _ in range(2)],
            [pltpu.VMEM((CH_L, NB), jnp.int32) for _ in range(2)],
            [pltpu.VMEM((CH_L * NB, 128), jnp.float32) for _ in range(2)],
            [pltpu.VMEM((CH_L, 8, 8, NB), jnp.float32) for _ in range(2)],
            pltpu.VMEM((S // 2, 128), jnp.float32),
            [pltpu.SemaphoreType.DMA for _ in range(2)],
            [pltpu.SemaphoreType.DMA for _ in range(2)],
        ],
        compiler_params=pltpu.CompilerParams(
            use_tc_tiling_on_sc=True, needs_layout_passes=False),
    )
    def k(xT_hbm, table_hbm, pe_hbm, out_hbm, idx_v, idx2_v, h_v, rows_v,
          trans_v, pe_v, gsem, osem):
        wid = lax.axis_index("s") * NC + lax.axis_index("c")
        base = wid * NB
        pltpu.sync_copy(pe_hbm, pe_v)

        iota = lax.iota(jnp.int32, L)
        bvec = [iota + j * L for j in range(NB // L)]

        def load_idx(c, p):
            pltpu.sync_copy(
                xT_hbm.at[pl.ds(c * CH_L, CH_L), pl.ds(base, NB)], idx_v[p])
            for i in range(CH_L):
                for j in range(NB // L):
                    t = idx_v[p][i, pl.ds(j * L, L)]
                    idx2_v[p][i, pl.ds(j * L, L)] = t >> 1
                    h_v[p][i, pl.ds(j * L, L)] = (t & 1) << 6

        def fire_gathers(p):
            for i in range(CH_L):
                pltpu.async_copy(
                    table_hbm.at[idx2_v[p].at[i]],
                    rows_v[p].at[pl.ds(i * NB, NB)], gsem[p])

        def wait_gathers(p):
            for i in range(CH_L):
                pltpu.make_async_copy(
                    table_hbm.at[idx2_v[p].at[i]],
                    rows_v[p].at[pl.ds(i * NB, NB)], gsem[p]).wait()

        def out_slab(c):
            return out_hbm.at[pl.ds(c * CH_L, CH_L), :, wid, :, :]

        def vec_process(c, p):
            for i in range(CH_L):
                h16 = [h_v[p][i, pl.ds(j * L, L)] for j in range(NB // L)]
                rowv = [bvec[j] + i * NB for j in range(NB // L)]
                pc0 = (i & 1) * D
                per_splat = jnp.broadcast_to(c, (L,))
                ivec = jnp.full((L,), i, dtype=jnp.int32)
                for d0 in range(0, D, L):

                    @plsc.parallel_loop(0, L, unroll=2)
                    def _m(m):
                        gcol = ((iota + m) & (L - 1)) + d0
                        pvec = plsc.load_gather(
                            pe_v, [per_splat, gcol + pc0])
                        rvec = gcol >> 3
                        drvec = gcol & 7
                        for j in range(NB // L):
                            g16 = plsc.load_gather(
                                rows_v[p], [rowv[j], h16[j] + gcol])
                            plsc.store_scatter(
                                trans_v[p],
                                [ivec, rvec, drvec, bvec[j]], g16 + pvec)

        def fire_out(c, p):
            pltpu.async_copy(trans_v[p], out_slab(c), osem[p])

        def wait_out(c, p):
            pltpu.make_async_copy(trans_v[p], out_slab(c), osem[p]).wait()

        load_idx(0, 0)
        fire_gathers(0)
        load_idx(1, 1)
        fire_gathers(1)

        @pl.loop(0, NPAIR)
        def _pair(kk):
            a = kk * 2
            b = a + 1

            wait_gathers(0)

            @pl.when(kk > 0)
            def _():
                wait_out(a, 0)

            vec_process(a, 0)
            fire_out(a, 0)

            @pl.when(kk < NPAIR - 1)
            def _():
                load_idx(a + 2, 0)
                fire_gathers(0)

            wait_gathers(1)

            @pl.when(kk > 0)
            def _():
                wait_out(b, 1)

            vec_process(b, 1)
            fire_out(b, 1)

            @pl.when(kk < NPAIR - 1)
            def _():
                load_idx(b + 2, 1)
                fire_gathers(1)

        wait_out(NCHUNK - 2, 0)
        wait_out(NCHUNK - 1, 1)

    return k(xT, table2, pe2)


@jax.jit
def kernel(x, table, pe):
    xT = x.T.astype(jnp.int32)
    table2 = jnp.concatenate([table[0::2], table[1::2]], axis=1)
    pe2 = pe[0, :S, :].reshape(S // 2, 128)
    res = _sc_embed(xT, table2, pe2)
    return jnp.transpose(res, (2, 4, 0, 1, 3)).reshape(B, S, D)


# speedup vs baseline: 7.2467x; 7.2467x over previous
import functools

import jax
import jax.numpy as jnp
from jax import lax
from jax.experimental import pallas as pl
from jax.experimental.pallas import tpu as pltpu
from jax.experimental.pallas import tpu_sc as plsc

B = 4096
S = 200
D = 64
L = 16

NC = 2
NS = 16
NW = NC * NS

NB = B // NW
CH_L = 2
NCHUNK = S // CH_L
NPAIR = NCHUNK // 2


def _sc_embed(xT, table2, pe2):
    mesh = plsc.VectorSubcoreMesh(
        core_axis_name="c", subcore_axis_name="s", num_cores=NC,
        num_subcores=NS)

    @functools.partial(
        pl.kernel,
        mesh=mesh,
        out_type=jax.ShapeDtypeStruct((S, 8, NW, 8, NB), jnp.float32),
        scratch_types=[
            [pltpu.VMEM((CH_L, NB), jnp.int32) for _ in range(2)],
            [pltpu.VMEM((CH_L * NB,), jnp.int32) for _ in range(2)],
            [pltpu.VMEM((CH_L * NB,), jnp.int32) for _ in range(2)],
            [pltpu.VMEM((CH_L * NB, 128), jnp.float32) for _ in range(2)],
            [pltpu.VMEM((CH_L, 8, 8, NB), jnp.float32) for _ in range(2)],
            pltpu.VMEM((S // 2, 128), jnp.float32),
            [pltpu.SemaphoreType.DMA for _ in range(2)],
            [pltpu.SemaphoreType.DMA for _ in range(2)],
        ],
        compiler_params=pltpu.CompilerParams(
            use_tc_tiling_on_sc=True, needs_layout_passes=False),
    )
    def k(xT_hbm, table_hbm, pe_hbm, out_hbm, idx_v, idx2_v, h_v, rows_v,
          trans_v, pe_v, gsem, osem):
        wid = lax.axis_index("s") * NC + lax.axis_index("c")
        base = wid * NB
        pltpu.sync_copy(pe_hbm, pe_v)

        iota = lax.iota(jnp.int32, L)
        bvec = [iota + j * L for j in range(NB // L)]

        def load_idx(c, p):
            pltpu.sync_copy(
                xT_hbm.at[pl.ds(c * CH_L, CH_L), pl.ds(base, NB)], idx_v[p])
            for i in range(CH_L):
                for j in range(NB // L):
                    t = idx_v[p][i, pl.ds(j * L, L)]
                    idx2_v[p][pl.ds(i * NB + j * L, L)] = t >> 1
                    h_v[p][pl.ds(i * NB + j * L, L)] = (t & 1) << 6

        def fire_gathers(p):
            pltpu.async_copy(table_hbm.at[idx2_v[p]], rows_v[p], gsem[p])

        def wait_gathers(p):
            pltpu.make_async_copy(
                table_hbm.at[idx2_v[p]], rows_v[p], gsem[p]).wait()

        def out_slab(c):
            return out_hbm.at[pl.ds(c * CH_L, CH_L), :, wid, :, :]

        def vec_process(c, p):
            for i in range(CH_L):
                h16 = [h_v[p][pl.ds(i * NB + j * L, L)]
                       for j in range(NB // L)]
                rowv = [bvec[j] + i * NB for j in range(NB // L)]
                pc0 = (i & 1) * D
                per_splat = jnp.broadcast_to(c, (L,))
                ivec = jnp.full((L,), i, dtype=jnp.int32)
                for d0 in range(0, D, L):

                    @plsc.parallel_loop(0, L, unroll=4)
                    def _m(m):
                        gcol = ((iota + m) & (L - 1)) + d0
                        pvec = plsc.load_gather(
                            pe_v, [per_splat, gcol + pc0])
                        rvec = gcol >> 3
                        drvec = gcol & 7
                        for j in range(NB // L):
                            g16 = plsc.load_gather(
                                rows_v[p], [rowv[j], h16[j] + gcol])
                            plsc.store_scatter(
                                trans_v[p],
                                [ivec, rvec, drvec, bvec[j]], g16 + pvec)

        def fire_out(c, p):
            pltpu.async_copy(trans_v[p], out_slab(c), osem[p])

        def wait_out(c, p):
            pltpu.make_async_copy(trans_v[p], out_slab(c), osem[p]).wait()

        load_idx(0, 0)
        fire_gathers(0)
        load_idx(1, 1)
        fire_gathers(1)

        @pl.loop(0, NPAIR)
        def _pair(kk):
            a = kk * 2
            b = a + 1

            wait_gathers(0)

            @pl.when(kk > 0)
            def _():
                wait_out(a, 0)

            vec_process(a, 0)
            fire_out(a, 0)

            @pl.when(kk < NPAIR - 1)
            def _():
                load_idx(a + 2, 0)
                fire_gathers(0)

            wait_gathers(1)

            @pl.when(kk > 0)
            def _():
                wait_out(b, 1)

            vec_process(b, 1)
            fire_out(b, 1)

            @pl.when(kk < NPAIR - 1)
            def _():
                load_idx(b + 2, 1)
                fire_gathers(1)

        wait_out(NCHUNK - 2, 0)
        wait_out(NCHUNK - 1, 1)

    return k(xT, table2, pe2)


@jax.jit
def kernel(x, table, pe):
    xT = x.T.astype(jnp.int32)
    table2 = table.reshape(500000, 128)
    pe2 = pe[0, :S, :].reshape(S // 2, 128)
    res = _sc_embed(xT, table2, pe2)
    return jnp.transpose(res, (2, 4, 0, 1, 3)).reshape(B, S, D)


# speedup vs baseline: 9.2942x; 1.2826x over previous
import functools

import jax
import jax.numpy as jnp
from jax import lax
from jax.experimental import pallas as pl
from jax.experimental.pallas import tpu as pltpu
from jax.experimental.pallas import tpu_sc as plsc

B = 4096
S = 200
D = 64
L = 16

NC = 2
NS = 16
NW = NC * NS

NB = B // NW
CH_L = 2
NCHUNK = S // CH_L
NPAIR = NCHUNK // 2


def _sc_embed(xT, table2, pe2):
    mesh = plsc.VectorSubcoreMesh(
        core_axis_name="c", subcore_axis_name="s", num_cores=NC,
        num_subcores=NS)

    @functools.partial(
        pl.kernel,
        mesh=mesh,
        out_type=jax.ShapeDtypeStruct((S, 8, NW, 8, NB), jnp.float32),
        scratch_types=[
            [pltpu.VMEM((CH_L, NB), jnp.int32) for _ in range(2)],
            [pltpu.VMEM((CH_L * NB,), jnp.int32) for _ in range(2)],
            [pltpu.VMEM((CH_L * NB,), jnp.int32) for _ in range(2)],
            [pltpu.VMEM((CH_L * NB, 128), jnp.float32) for _ in range(2)],
            [pltpu.VMEM((CH_L, 8, 8, NB), jnp.float32) for _ in range(2)],
            pltpu.VMEM((S // 2, 128), jnp.float32),
            [pltpu.SemaphoreType.DMA for _ in range(2)],
            [pltpu.SemaphoreType.DMA for _ in range(2)],
        ],
        compiler_params=pltpu.CompilerParams(
            use_tc_tiling_on_sc=True, needs_layout_passes=False),
    )
    def k(xT_hbm, table_hbm, pe_hbm, out_hbm, idx_v, idx2_v, h_v, rows_v,
          trans_v, pe_v, gsem, osem):
        wid = lax.axis_index("s") * NC + lax.axis_index("c")
        base = wid * NB
        pltpu.sync_copy(pe_hbm, pe_v)

        iota = lax.iota(jnp.int32, L)
        bvec = [iota + j * L for j in range(NB // L)]

        def load_idx(c, p):
            pltpu.sync_copy(
                xT_hbm.at[pl.ds(c * CH_L, CH_L), pl.ds(base, NB)], idx_v[p])
            for i in range(CH_L):
                for j in range(NB // L):
                    t = idx_v[p][i, pl.ds(j * L, L)]
                    idx2_v[p][pl.ds(i * NB + j * L, L)] = t >> 1
                    h_v[p][pl.ds(i * NB + j * L, L)] = (t & 1) << 6

        def fire_gathers(p):
            pltpu.async_copy(table_hbm.at[idx2_v[p]], rows_v[p], gsem[p])

        def wait_gathers(p):
            pltpu.make_async_copy(
                table_hbm.at[idx2_v[p]], rows_v[p], gsem[p]).wait()

        def out_slab(c):
            return out_hbm.at[pl.ds(c * CH_L, CH_L), :, wid, :, :]

        def vec_process(c, p):
            for i in range(CH_L):
                h16 = [h_v[p][pl.ds(i * NB + j * L, L)]
                       for j in range(NB // L)]
                rowv = [bvec[j] + i * NB for j in range(NB // L)]
                pc0 = (i & 1) * D
                per_splat = jnp.broadcast_to(c, (L,))
                ivec = jnp.full((L,), i, dtype=jnp.int32)
                for d0 in range(0, D, L):

                    @plsc.parallel_loop(0, L, unroll=2)
                    def _m(m):
                        gcol = ((iota + m) & (L - 1)) + d0
                        pvec = plsc.load_gather(
                            pe_v, [per_splat, gcol + pc0])
                        rvec = gcol >> 3
                        drvec = gcol & 7
                        for j in range(NB // L):
                            g16 = plsc.load_gather(
                                rows_v[p], [rowv[j], h16[j] + gcol])
                            plsc.store_scatter(
                                trans_v[p],
                                [ivec, rvec, drvec, bvec[j]], g16 + pvec)

        def fire_out(c, p):
            pltpu.async_copy(trans_v[p], out_slab(c), osem[p])

        def wait_out(c, p):
            pltpu.make_async_copy(trans_v[p], out_slab(c), osem[p]).wait()

        load_idx(0, 0)
        fire_gathers(0)
        load_idx(1, 1)
        fire_gathers(1)

        @pl.loop(0, NPAIR)
        def _pair(kk):
            a = kk * 2
            b = a + 1

            wait_gathers(0)

            @pl.when(kk > 0)
            def _():
                wait_out(a, 0)

            vec_process(a, 0)
            fire_out(a, 0)

            @pl.when(kk < NPAIR - 1)
            def _():
                load_idx(a + 2, 0)
                fire_gathers(0)

            wait_gathers(1)

            @pl.when(kk > 0)
            def _():
                wait_out(b, 1)

            vec_process(b, 1)
            fire_out(b, 1)

            @pl.when(kk < NPAIR - 1)
            def _():
                load_idx(b + 2, 1)
                fire_gathers(1)

        wait_out(NCHUNK - 2, 0)
        wait_out(NCHUNK - 1, 1)

    return k(xT, table2, pe2)


@jax.jit
def kernel(x, table, pe):
    xT = x.T.astype(jnp.int32)
    table2 = table.reshape(500000, 128)
    pe2 = pe[0, :S, :].reshape(S // 2, 128)
    res = _sc_embed(xT, table2, pe2)
    return jnp.transpose(res, (2, 4, 0, 1, 3)).reshape(B, S, D)


# speedup vs baseline: 9.8467x; 1.0594x over previous
import functools

import jax
import jax.numpy as jnp
from jax import lax
from jax.experimental import pallas as pl
from jax.experimental.pallas import tpu as pltpu
from jax.experimental.pallas import tpu_sc as plsc

B = 4096
S = 200
D = 64
L = 16

NC = 2
NS = 16
NW = NC * NS

NB = B // NW
CH_L = 2
NCHUNK = S // CH_L
NPAIR = NCHUNK // 2


TRB = 2048


def _tr_body(x_ref, o_ref):
    xt = x_ref[...].T.reshape(TRB // 2, 2, 64)
    o_ref[:, 0:64] = xt[:, 0, :]
    o_ref[:, 64:128] = xt[:, 1, :]


def _tc_rowmajor(tableT):
    return pl.pallas_call(
        _tr_body,
        out_shape=jax.ShapeDtypeStruct((500000, 128), jnp.float32),
        grid=(pl.cdiv(1000000, TRB),),
        in_specs=[pl.BlockSpec((64, TRB), lambda i: (0, i))],
        out_specs=pl.BlockSpec((TRB // 2, 128), lambda i: (i, 0)),
    )(tableT)


def _sc_embed(xT, table2, pe2):
    mesh = plsc.VectorSubcoreMesh(
        core_axis_name="c", subcore_axis_name="s", num_cores=NC,
        num_subcores=NS)

    @functools.partial(
        pl.kernel,
        mesh=mesh,
        out_type=jax.ShapeDtypeStruct((S, 8, NW, 8, NB), jnp.float32),
        scratch_types=[
            [pltpu.VMEM((CH_L, NB), jnp.int32) for _ in range(2)],
            [pltpu.VMEM((CH_L * NB,), jnp.int32) for _ in range(2)],
            [pltpu.VMEM((CH_L * NB,), jnp.int32) for _ in range(2)],
            [pltpu.VMEM((CH_L * NB, 128), jnp.float32) for _ in range(2)],
            [pltpu.VMEM((CH_L, 8, 8, NB), jnp.float32) for _ in range(2)],
            pltpu.VMEM((S // 2, 128), jnp.float32),
            [pltpu.SemaphoreType.DMA for _ in range(2)],
            [pltpu.SemaphoreType.DMA for _ in range(2)],
        ],
        compiler_params=pltpu.CompilerParams(
            use_tc_tiling_on_sc=True, needs_layout_passes=False),
    )
    def k(xT_hbm, table_hbm, pe_hbm, out_hbm, idx_v, idx2_v, h_v, rows_v,
          trans_v, pe_v, gsem, osem):
        wid = lax.axis_index("s") * NC + lax.axis_index("c")
        base = wid * NB
        pltpu.sync_copy(pe_hbm, pe_v)

        iota = lax.iota(jnp.int32, L)
        bvec = [iota + j * L for j in range(NB // L)]

        def load_idx(c, p):
            pltpu.sync_copy(
                xT_hbm.at[pl.ds(c * CH_L, CH_L), pl.ds(base, NB)], idx_v[p])
            for i in range(CH_L):
                for j in range(NB // L):
                    t = idx_v[p][i, pl.ds(j * L, L)]
                    idx2_v[p][pl.ds(i * NB + j * L, L)] = t >> 1
                    h_v[p][pl.ds(i * NB + j * L, L)] = (t & 1) << 6

        def fire_gathers(p):
            pltpu.async_copy(table_hbm.at[idx2_v[p]], rows_v[p], gsem[p])

        def wait_gathers(p):
            pltpu.make_async_copy(
                table_hbm.at[idx2_v[p]], rows_v[p], gsem[p]).wait()

        def out_slab(c):
            return out_hbm.at[pl.ds(c * CH_L, CH_L), :, wid, :, :]

        def vec_process(c, p):
            for i in range(CH_L):
                h16 = [h_v[p][pl.ds(i * NB + j * L, L)]
                       for j in range(NB // L)]
                rowv = [bvec[j] + i * NB for j in range(NB // L)]
                pc0 = (i & 1) * D
                per_splat = jnp.broadcast_to(c, (L,))
                ivec = jnp.full((L,), i, dtype=jnp.int32)
                for d0 in range(0, D, L):

                    @plsc.parallel_loop(0, L, unroll=2)
                    def _m(m):
                        gcol = ((iota + m) & (L - 1)) + d0
                        pvec = plsc.load_gather(
                            pe_v, [per_splat, gcol + pc0])
                        rvec = gcol >> 3
                        drvec = gcol & 7
                        for j in range(NB // L):
                            g16 = plsc.load_gather(
                                rows_v[p], [rowv[j], h16[j] + gcol])
                            plsc.store_scatter(
                                trans_v[p],
                                [ivec, rvec, drvec, bvec[j]], g16 + pvec)

        def fire_out(c, p):
            pltpu.async_copy(trans_v[p], out_slab(c), osem[p])

        def wait_out(c, p):
            pltpu.make_async_copy(trans_v[p], out_slab(c), osem[p]).wait()

        load_idx(0, 0)
        fire_gathers(0)
        load_idx(1, 1)
        fire_gathers(1)

        @pl.loop(0, NPAIR)
        def _pair(kk):
            a = kk * 2
            b = a + 1

            wait_gathers(0)

            @pl.when(kk > 0)
            def _():
                wait_out(a, 0)

            vec_process(a, 0)
            fire_out(a, 0)

            @pl.when(kk < NPAIR - 1)
            def _():
                load_idx(a + 2, 0)
                fire_gathers(0)

            wait_gathers(1)

            @pl.when(kk > 0)
            def _():
                wait_out(b, 1)

            vec_process(b, 1)
            fire_out(b, 1)

            @pl.when(kk < NPAIR - 1)
            def _():
                load_idx(b + 2, 1)
                fire_gathers(1)

        wait_out(NCHUNK - 2, 0)
        wait_out(NCHUNK - 1, 1)

    return k(xT, table2, pe2)


@jax.jit
def kernel(x, table, pe):
    xT = x.T.astype(jnp.int32)
    table2 = _tc_rowmajor(table.T)
    pe2 = pe[0, :S, :].reshape(S // 2, 128)
    res = _sc_embed(xT, table2, pe2)
    return jnp.transpose(res, (2, 4, 0, 1, 3)).reshape(B, S, D)


# speedup vs baseline: 11.2996x; 1.1476x over previous
import functools

import jax
import jax.numpy as jnp
from jax import lax
from jax.experimental import pallas as pl
from jax.experimental.pallas import tpu as pltpu
from jax.experimental.pallas import tpu_sc as plsc

B = 4096
S = 200
D = 64
L = 16

NC = 2
NS = 16
NW = NC * NS

NB = B // NW
CH_L = 2
NCHUNK = S // CH_L
NPAIR = NCHUNK // 2


TRB = 4096


def _tr_body(x_ref, o_ref):
    xt = x_ref[...].T.reshape(TRB // 2, 2, 64)
    o_ref[:, 0:64] = xt[:, 0, :]
    o_ref[:, 64:128] = xt[:, 1, :]


def _tc_rowmajor(tableT):
    return pl.pallas_call(
        _tr_body,
        out_shape=jax.ShapeDtypeStruct((500000, 128), jnp.float32),
        grid=(pl.cdiv(1000000, TRB),),
        in_specs=[pl.BlockSpec((64, TRB), lambda i: (0, i))],
        out_specs=pl.BlockSpec((TRB // 2, 128), lambda i: (i, 0)),
    )(tableT)


def _sc_embed(xT, table2, pe2):
    mesh = plsc.VectorSubcoreMesh(
        core_axis_name="c", subcore_axis_name="s", num_cores=NC,
        num_subcores=NS)

    @functools.partial(
        pl.kernel,
        mesh=mesh,
        out_type=jax.ShapeDtypeStruct((S, 8, NW, 8, NB), jnp.float32),
        scratch_types=[
            [pltpu.VMEM((CH_L, NB), jnp.int32) for _ in range(2)],
            [pltpu.VMEM((CH_L * NB,), jnp.int32) for _ in range(2)],
            [pltpu.VMEM((CH_L * NB,), jnp.int32) for _ in range(2)],
            [pltpu.VMEM((CH_L * NB, 128), jnp.float32) for _ in range(2)],
            [pltpu.VMEM((CH_L, 8, 8, NB), jnp.float32) for _ in range(2)],
            pltpu.VMEM((S // 2, 128), jnp.float32),
            [pltpu.SemaphoreType.DMA for _ in range(2)],
            [pltpu.SemaphoreType.DMA for _ in range(2)],
        ],
        compiler_params=pltpu.CompilerParams(
            use_tc_tiling_on_sc=True, needs_layout_passes=False),
    )
    def k(xT_hbm, table_hbm, pe_hbm, out_hbm, idx_v, idx2_v, h_v, rows_v,
          trans_v, pe_v, gsem, osem):
        wid = lax.axis_index("s") * NC + lax.axis_index("c")
        base = wid * NB
        pltpu.sync_copy(pe_hbm, pe_v)

        iota = lax.iota(jnp.int32, L)
        bvec = [iota + j * L for j in range(NB // L)]

        def load_idx(c, p):
            pltpu.sync_copy(
                xT_hbm.at[pl.ds(c * CH_L, CH_L), pl.ds(base, NB)], idx_v[p])
            for i in range(CH_L):
                for j in range(NB // L):
                    t = idx_v[p][i, pl.ds(j * L, L)]
                    idx2_v[p][pl.ds(i * NB + j * L, L)] = t >> 1
                    h_v[p][pl.ds(i * NB + j * L, L)] = (t & 1) << 6

        def fire_gathers(p):
            pltpu.async_copy(table_hbm.at[idx2_v[p]], rows_v[p], gsem[p])

        def wait_gathers(p):
            pltpu.make_async_copy(
                table_hbm.at[idx2_v[p]], rows_v[p], gsem[p]).wait()

        def out_slab(c):
            return out_hbm.at[pl.ds(c * CH_L, CH_L), :, wid, :, :]

        def vec_process(c, p):
            for i in range(CH_L):
                h16 = [h_v[p][pl.ds(i * NB + j * L, L)]
                       for j in range(NB // L)]
                rowv = [bvec[j] + i * NB for j in range(NB // L)]
                pc0 = (i & 1) * D
                per_splat = jnp.broadcast_to(c, (L,))
                ivec = jnp.full((L,), i, dtype=jnp.int32)
                for d0 in range(0, D, L):

                    @plsc.parallel_loop(0, L, unroll=2)
                    def _m(m):
                        gcol = ((iota + m) & (L - 1)) + d0
                        pvec = plsc.load_gather(
                            pe_v, [per_splat, gcol + pc0])
                        rvec = gcol >> 3
                        drvec = gcol & 7
                        for j in range(NB // L):
                            g16 = plsc.load_gather(
                                rows_v[p], [rowv[j], h16[j] + gcol])
                            plsc.store_scatter(
                                trans_v[p],
                                [ivec, rvec, drvec, bvec[j]], g16 + pvec)

        def fire_out(c, p):
            pltpu.async_copy(trans_v[p], out_slab(c), osem[p])

        def wait_out(c, p):
            pltpu.make_async_copy(trans_v[p], out_slab(c), osem[p]).wait()

        load_idx(0, 0)
        fire_gathers(0)
        load_idx(1, 1)
        fire_gathers(1)

        @pl.loop(0, NPAIR)
        def _pair(kk):
            a = kk * 2
            b = a + 1

            wait_gathers(0)

            @pl.when(kk > 0)
            def _():
                wait_out(a, 0)

            vec_process(a, 0)
            fire_out(a, 0)

            @pl.when(kk < NPAIR - 1)
            def _():
                load_idx(a + 2, 0)
                fire_gathers(0)

            wait_gathers(1)

            @pl.when(kk > 0)
            def _():
                wait_out(b, 1)

            vec_process(b, 1)
            fire_out(b, 1)

            @pl.when(kk < NPAIR - 1)
            def _():
                load_idx(b + 2, 1)
                fire_gathers(1)

        wait_out(NCHUNK - 2, 0)
        wait_out(NCHUNK - 1, 1)

    return k(xT, table2, pe2)


@jax.jit
def kernel(x, table, pe):
    xT = x.T.astype(jnp.int32)
    table2 = _tc_rowmajor(table.T)
    pe2 = pe[0, :S, :].reshape(S // 2, 128)
    res = _sc_embed(xT, table2, pe2)
    return jnp.transpose(res, (2, 4, 0, 1, 3)).reshape(B, S, D)


# speedup vs baseline: 11.7540x; 1.0402x over previous
import functools

import jax
import jax.numpy as jnp
from jax import lax
from jax.experimental import pallas as pl
from jax.experimental.pallas import tpu as pltpu
from jax.experimental.pallas import tpu_sc as plsc

B = 4096
S = 200
D = 64
L = 16

NC = 2
NS = 16
NW = NC * NS

NB = B // NW
CH_L = 2
NCHUNK = S // CH_L
NPAIR = NCHUNK // 2


TRB = 8192


def _tr_body(x_ref, o_ref):
    xt = x_ref[...].T.reshape(TRB // 2, 2, 64)
    o_ref[:, 0:64] = xt[:, 0, :]
    o_ref[:, 64:128] = xt[:, 1, :]


def _tc_rowmajor(tableT):
    return pl.pallas_call(
        _tr_body,
        out_shape=jax.ShapeDtypeStruct((500000, 128), jnp.float32),
        grid=(pl.cdiv(1000000, TRB),),
        in_specs=[pl.BlockSpec((64, TRB), lambda i: (0, i))],
        out_specs=pl.BlockSpec((TRB // 2, 128), lambda i: (i, 0)),
    )(tableT)


def _sc_embed(xT, table2, pe2):
    mesh = plsc.VectorSubcoreMesh(
        core_axis_name="c", subcore_axis_name="s", num_cores=NC,
        num_subcores=NS)

    @functools.partial(
        pl.kernel,
        mesh=mesh,
        out_type=jax.ShapeDtypeStruct((S, 8, NW, 8, NB), jnp.float32),
        scratch_types=[
            [pltpu.VMEM((CH_L, NB), jnp.int32) for _ in range(2)],
            [pltpu.VMEM((CH_L * NB,), jnp.int32) for _ in range(2)],
            [pltpu.VMEM((CH_L * NB,), jnp.int32) for _ in range(2)],
            [pltpu.VMEM((CH_L * NB, 128), jnp.float32) for _ in range(2)],
            [pltpu.VMEM((CH_L, 8, 8, NB), jnp.float32) for _ in range(2)],
            pltpu.VMEM((S // 2, 128), jnp.float32),
            [pltpu.SemaphoreType.DMA for _ in range(2)],
            [pltpu.SemaphoreType.DMA for _ in range(2)],
        ],
        compiler_params=pltpu.CompilerParams(
            use_tc_tiling_on_sc=True, needs_layout_passes=False),
    )
    def k(xT_hbm, table_hbm, pe_hbm, out_hbm, idx_v, idx2_v, h_v, rows_v,
          trans_v, pe_v, gsem, osem):
        wid = lax.axis_index("s") * NC + lax.axis_index("c")
        base = wid * NB
        pltpu.sync_copy(pe_hbm, pe_v)

        iota = lax.iota(jnp.int32, L)
        bvec = [iota + j * L for j in range(NB // L)]

        def load_idx(c, p):
            pltpu.sync_copy(
                xT_hbm.at[pl.ds(c * CH_L, CH_L), pl.ds(base, NB)], idx_v[p])
            for i in range(CH_L):
                for j in range(NB // L):
                    t = idx_v[p][i, pl.ds(j * L, L)]
                    idx2_v[p][pl.ds(i * NB + j * L, L)] = t >> 1
                    h_v[p][pl.ds(i * NB + j * L, L)] = (t & 1) << 6

        def fire_gathers(p):
            pltpu.async_copy(table_hbm.at[idx2_v[p]], rows_v[p], gsem[p])

        def wait_gathers(p):
            pltpu.make_async_copy(
                table_hbm.at[idx2_v[p]], rows_v[p], gsem[p]).wait()

        def out_slab(c):
            return out_hbm.at[pl.ds(c * CH_L, CH_L), :, wid, :, :]

        def vec_process(c, p):
            for i in range(CH_L):
                h16 = [h_v[p][pl.ds(i * NB + j * L, L)]
                       for j in range(NB // L)]
                rowv = [bvec[j] + i * NB for j in range(NB // L)]
                pc0 = (i & 1) * D
                per_splat = jnp.broadcast_to(c, (L,))
                ivec = jnp.full((L,), i, dtype=jnp.int32)
                for d0 in range(0, D, L):

                    @plsc.parallel_loop(0, L, unroll=2)
                    def _m(m):
                        gcol = ((iota + m) & (L - 1)) + d0
                        pvec = plsc.load_gather(
                            pe_v, [per_splat, gcol + pc0])
                        rvec = gcol >> 3
                        drvec = gcol & 7
                        for j in range(NB // L):
                            g16 = plsc.load_gather(
                                rows_v[p], [rowv[j], h16[j] + gcol])
                            plsc.store_scatter(
                                trans_v[p],
                                [ivec, rvec, drvec, bvec[j]], g16 + pvec)

        def fire_out(c, p):
            pltpu.async_copy(trans_v[p], out_slab(c), osem[p])

        def wait_out(c, p):
            pltpu.make_async_copy(trans_v[p], out_slab(c), osem[p]).wait()

        load_idx(0, 0)
        fire_gathers(0)
        load_idx(1, 1)
        fire_gathers(1)

        @pl.loop(0, NPAIR)
        def _pair(kk):
            a = kk * 2
            b = a + 1

            wait_gathers(0)

            @pl.when(kk > 0)
            def _():
                wait_out(a, 0)

            vec_process(a, 0)
            fire_out(a, 0)

            @pl.when(kk < NPAIR - 1)
            def _():
                load_idx(a + 2, 0)
                fire_gathers(0)

            wait_gathers(1)

            @pl.when(kk > 0)
            def _():
                wait_out(b, 1)

            vec_process(b, 1)
            fire_out(b, 1)

            @pl.when(kk < NPAIR - 1)
            def _():
                load_idx(b + 2, 1)
                fire_gathers(1)

        wait_out(NCHUNK - 2, 0)
        wait_out(NCHUNK - 1, 1)

    return k(xT, table2, pe2)


@jax.jit
def kernel(x, table, pe):
    xT = x.T.astype(jnp.int32)
    table2 = _tc_rowmajor(table.T)
    pe2 = pe[0, :S, :].reshape(S // 2, 128)
    res = _sc_embed(xT, table2, pe2)
    return jnp.transpose(res, (2, 4, 0, 1, 3)).reshape(B, S, D)


# speedup vs baseline: 11.8778x; 1.0105x over previous
import functools

import jax
import jax.numpy as jnp
from jax import lax
from jax.experimental import pallas as pl
from jax.experimental.pallas import tpu as pltpu
from jax.experimental.pallas import tpu_sc as plsc

B = 4096
S = 200
D = 64
L = 16

NC = 2
NS = 16
NW = NC * NS

NB = B // NW
CH_L = 2
NCHUNK = S // CH_L
NPAIR = NCHUNK // 2


TRB = 16384


def _tr_body(x_ref, o_ref):
    xt = x_ref[...].T.reshape(TRB // 2, 2, 64)
    o_ref[:, 0:64] = xt[:, 0, :]
    o_ref[:, 64:128] = xt[:, 1, :]


def _tc_rowmajor(tableT):
    return pl.pallas_call(
        _tr_body,
        out_shape=jax.ShapeDtypeStruct((500000, 128), jnp.float32),
        grid=(pl.cdiv(1000000, TRB),),
        in_specs=[pl.BlockSpec((64, TRB), lambda i: (0, i))],
        out_specs=pl.BlockSpec((TRB // 2, 128), lambda i: (i, 0)),
    )(tableT)


def _sc_embed(xT, table2, pe2):
    mesh = plsc.VectorSubcoreMesh(
        core_axis_name="c", subcore_axis_name="s", num_cores=NC,
        num_subcores=NS)

    @functools.partial(
        pl.kernel,
        mesh=mesh,
        out_type=jax.ShapeDtypeStruct((S, 8, NW, 8, NB), jnp.float32),
        scratch_types=[
            [pltpu.VMEM((CH_L, NB), jnp.int32) for _ in range(2)],
            [pltpu.VMEM((CH_L * NB,), jnp.int32) for _ in range(2)],
            [pltpu.VMEM((CH_L * NB,), jnp.int32) for _ in range(2)],
            [pltpu.VMEM((CH_L * NB, 128), jnp.float32) for _ in range(2)],
            [pltpu.VMEM((CH_L, 8, 8, NB), jnp.float32) for _ in range(2)],
            pltpu.VMEM((S // 2, 128), jnp.float32),
            [pltpu.SemaphoreType.DMA for _ in range(2)],
            [pltpu.SemaphoreType.DMA for _ in range(2)],
        ],
        compiler_params=pltpu.CompilerParams(
            use_tc_tiling_on_sc=True, needs_layout_passes=False),
    )
    def k(xT_hbm, table_hbm, pe_hbm, out_hbm, idx_v, idx2_v, h_v, rows_v,
          trans_v, pe_v, gsem, osem):
        wid = lax.axis_index("s") * NC + lax.axis_index("c")
        base = wid * NB
        pltpu.sync_copy(pe_hbm, pe_v)

        iota = lax.iota(jnp.int32, L)
        bvec = [iota + j * L for j in range(NB // L)]

        def load_idx(c, p):
            pltpu.sync_copy(
                xT_hbm.at[pl.ds(c * CH_L, CH_L), pl.ds(base, NB)], idx_v[p])
            for i in range(CH_L):
                for j in range(NB // L):
                    t = idx_v[p][i, pl.ds(j * L, L)]
                    idx2_v[p][pl.ds(i * NB + j * L, L)] = t >> 1
                    h_v[p][pl.ds(i * NB + j * L, L)] = (t & 1) << 6

        def fire_gathers(p):
            pltpu.async_copy(table_hbm.at[idx2_v[p]], rows_v[p], gsem[p])

        def wait_gathers(p):
            pltpu.make_async_copy(
                table_hbm.at[idx2_v[p]], rows_v[p], gsem[p]).wait()

        def out_slab(c):
            return out_hbm.at[pl.ds(c * CH_L, CH_L), :, wid, :, :]

        def vec_process(c, p):
            for i in range(CH_L):
                h16 = [h_v[p][pl.ds(i * NB + j * L, L)]
                       for j in range(NB // L)]
                rowv = [bvec[j] + i * NB for j in range(NB // L)]
                pc0 = (i & 1) * D
                per_splat = jnp.broadcast_to(c, (L,))
                ivec = jnp.full((L,), i, dtype=jnp.int32)
                for d0 in range(0, D, L):

                    @plsc.parallel_loop(0, L, unroll=2)
                    def _m(m):
                        gcol = ((iota + m) & (L - 1)) + d0
                        pvec = plsc.load_gather(
                            pe_v, [per_splat, gcol + pc0])
                        rvec = gcol >> 3
                        drvec = gcol & 7
                        for j in range(NB // L):
                            g16 = plsc.load_gather(
                                rows_v[p], [rowv[j], h16[j] + gcol])
                            plsc.store_scatter(
                                trans_v[p],
                                [ivec, rvec, drvec, bvec[j]], g16 + pvec)

        def fire_out(c, p):
            pltpu.async_copy(trans_v[p], out_slab(c), osem[p])

        def wait_out(c, p):
            pltpu.make_async_copy(trans_v[p], out_slab(c), osem[p]).wait()

        load_idx(0, 0)
        fire_gathers(0)
        load_idx(1, 1)
        fire_gathers(1)

        @pl.loop(0, NPAIR)
        def _pair(kk):
            a = kk * 2
            b = a + 1

            wait_gathers(0)

            @pl.when(kk > 0)
            def _():
                wait_out(a, 0)

            vec_process(a, 0)
            fire_out(a, 0)

            @pl.when(kk < NPAIR - 1)
            def _():
                load_idx(a + 2, 0)
                fire_gathers(0)

            wait_gathers(1)

            @pl.when(kk > 0)
            def _():
                wait_out(b, 1)

            vec_process(b, 1)
            fire_out(b, 1)

            @pl.when(kk < NPAIR - 1)
            def _():
                load_idx(b + 2, 1)
                fire_gathers(1)

        wait_out(NCHUNK - 2, 0)
        wait_out(NCHUNK - 1, 1)

    return k(xT, table2, pe2)


@jax.jit
def kernel(x, table, pe):
    xT = x.T.astype(jnp.int32)
    table2 = _tc_rowmajor(table.T)
    pe2 = pe[0, :S, :].reshape(S // 2, 128)
    res = _sc_embed(xT, table2, pe2)
    return jnp.transpose(res, (2, 4, 0, 1, 3)).reshape(B, S, D)
